# baseline (speedup 1.0000x reference)
import functools

import jax
import jax.numpy as jnp
from jax import lax
from jax.experimental import pallas as pl
from jax.experimental.pallas import tpu as pltpu
from jax.experimental.pallas import tpu_sc as plsc


_NC = 2
_NS = 16
_NW = _NC * _NS
_LANES = 16
_CH = 16
_NXB = 8
_DEPTH = 4


def _make_sc_add(b, lfull, ls, d):
    lpw = ls // _NW
    nch = lpw // _CH
    nsteps = nch * b
    nvec = d // _LANES
    mesh = plsc.VectorSubcoreMesh(core_axis_name="c", subcore_axis_name="s")

    @functools.partial(
        pl.kernel,
        out_type=jax.ShapeDtypeStruct((b * ls, d), jnp.float32),
        mesh=mesh,
        scratch_types=(
            [pltpu.VMEM((_CH, d), jnp.float32)] * (_NXB + 2)
            + [pltpu.SemaphoreType.DMA] * (2 * _NXB + 2)
        ),
    )
    def sc_add(x_hbm, table_hbm, out_hbm, *scratch):
        bufs = scratch[:_NXB]
        tbufs = scratch[_NXB:_NXB + 2]
        sxs = scratch[_NXB + 2:2 * _NXB + 2]
        sts = scratch[2 * _NXB + 2:2 * _NXB + 4]
        sos = scratch[2 * _NXB + 4:3 * _NXB + 4]
        cid = lax.axis_index("c")
        sid = lax.axis_index("s")
        wid = cid * _NS + sid
        tbase = wid * lpw

        def t_slice(i):
            return table_hbm.at[pl.ds(tbase + i * _CH, _CH)]

        def x_slice(i, bi):
            return x_hbm.at[pl.ds(bi * lfull + tbase + i * _CH, _CH)]

        def o_slice(i, bi):
            return out_hbm.at[pl.ds(bi * ls + tbase + i * _CH, _CH)]

        pltpu.async_copy(t_slice(0), tbufs[0], sts[0])
        pltpu.async_copy(t_slice(1), tbufs[1], sts[1])
        for k in range(_DEPTH):
            pltpu.async_copy(x_slice(k // b, k % b), bufs[k], sxs[k])

        def iter_body(i2, _):
            for ip in range(2):
                i = 2 * i2 + ip
                pltpu.make_async_copy(t_slice(i), tbufs[ip], sts[ip]).wait()
                for bi in range(b):
                    p = (ip * b + bi) % _NXB
                    s = i * b + bi
                    xb = bufs[p]
                    pltpu.make_async_copy(
                        x_slice(i, bi), xb, sxs[p]).wait()
                    np_ = (p + _DEPTH) % _NXB
                    nbi = (bi + _DEPTH) % b
                    ni = i + (bi + _DEPTH) // b

                    @pl.when(s + _DEPTH < nsteps)
                    def _():
                        @pl.when(s >= _NXB - _DEPTH)
                        def _():
                            pltpu.make_async_copy(
                                bufs[np_], o_slice(ni, nbi),
                                sos[np_]).wait()
                        pltpu.async_copy(
                            x_slice(ni, nbi), bufs[np_], sxs[np_])

                    tb = tbufs[ip]

                    @plsc.parallel_loop(0, _CH, step=1)
                    def add_body(r, xb=xb, tb=tb):
                        for c in range(nvec):
                            plsc.addupdate(
                                xb.at[r].at[pl.ds(c * _LANES, _LANES)],
                                tb[r, pl.ds(c * _LANES, _LANES)])

                    pltpu.async_copy(xb, o_slice(i, bi), sos[p])

                @pl.when(i + 2 < nch)
                def _():
                    pltpu.async_copy(t_slice(i + 2), tbufs[ip], sts[ip])
            return 0

        lax.fori_loop(0, nch // 2, iter_body, 0)
        for k in range(_NXB):
            sk = nsteps - _NXB + k
            pltpu.make_async_copy(
                bufs[sk % _NXB], o_slice(sk // b, sk % b),
                sos[sk % _NXB]).wait()

    return sc_add


def kernel(x, table):
    B, L, D = x.shape
    out = _make_sc_add(B, L, L, D)(x.reshape(B * L, D), table)
    return out.reshape(B, L, D)


# speedup vs baseline: 1.0081x; 1.0081x over previous
import functools

import jax
import jax.numpy as jnp
from jax import lax
from jax.experimental import pallas as pl
from jax.experimental.pallas import tpu as pltpu
from jax.experimental.pallas import tpu_sc as plsc


_NC = 2
_NS = 16
_NW = _NC * _NS
_LANES = 16
_CH = 16
_NXB = 4


def _make_sc_add(b, lfull, ls, d):
    lpw = ls // _NW
    nch = lpw // _CH
    nsteps = nch * b
    nvec = d // _LANES
    mesh = plsc.VectorSubcoreMesh(core_axis_name="c", subcore_axis_name="s")

    @functools.partial(
        pl.kernel,
        out_type=jax.ShapeDtypeStruct((b * ls, d), jnp.float32),
        mesh=mesh,
        scratch_types=(
            [pltpu.VMEM((_CH, d), jnp.float32)] * (_NXB + 2)
            + [pltpu.SemaphoreType.DMA] * (2 * _NXB + 2)
        ),
    )
    def sc_add(x_hbm, table_hbm, out_hbm, *scratch):
        bufs = scratch[:_NXB]
        tbufs = scratch[_NXB:_NXB + 2]
        sxs = scratch[_NXB + 2:2 * _NXB + 2]
        sts = scratch[2 * _NXB + 2:2 * _NXB + 4]
        sos = scratch[2 * _NXB + 4:3 * _NXB + 4]
        cid = lax.axis_index("c")
        sid = lax.axis_index("s")
        wid = cid * _NS + sid
        tbase = wid * lpw

        def t_slice(i):
            return table_hbm.at[pl.ds(tbase + i * _CH, _CH)]

        def x_slice(i, bi):
            return x_hbm.at[pl.ds(bi * lfull + tbase + i * _CH, _CH)]

        def o_slice(i, bi):
            return out_hbm.at[pl.ds(bi * ls + tbase + i * _CH, _CH)]

        pltpu.async_copy(t_slice(0), tbufs[0], sts[0])
        pltpu.async_copy(t_slice(1), tbufs[1], sts[1])
        pltpu.async_copy(x_slice(0, 0), bufs[0], sxs[0])
        pltpu.async_copy(x_slice(0, 1), bufs[1], sxs[1])

        def iter_body(i2, _):
            for ip in range(2):
                i = 2 * i2 + ip
                pltpu.make_async_copy(t_slice(i), tbufs[ip], sts[ip]).wait()
                for bi in range(b):
                    p = bi % _NXB
                    s = i * b + bi
                    xb = bufs[p]
                    pltpu.make_async_copy(
                        x_slice(i, bi), xb, sxs[p]).wait()
                    np_ = (p + 2) % _NXB
                    nbi = (bi + 2) % b
                    ni = i + (1 if bi >= b - 2 else 0)

                    @pl.when(s + 2 < nsteps)
                    def _():
                        @pl.when(s >= 2)
                        def _():
                            pltpu.make_async_copy(
                                bufs[np_], o_slice(ni, nbi),
                                sos[np_]).wait()
                        pltpu.async_copy(
                            x_slice(ni, nbi), bufs[np_], sxs[np_])

                    tb = tbufs[ip]

                    @plsc.parallel_loop(0, _CH, step=1)
                    def add_body(r, xb=xb, tb=tb):
                        for c in range(nvec):
                            plsc.addupdate(
                                xb.at[r].at[pl.ds(c * _LANES, _LANES)],
                                tb[r, pl.ds(c * _LANES, _LANES)])

                    pltpu.async_copy(xb, o_slice(i, bi), sos[p])

                @pl.when(i + 2 < nch)
                def _():
                    pltpu.async_copy(t_slice(i + 2), tbufs[ip], sts[ip])
            return 0

        lax.fori_loop(0, nch // 2, iter_body, 0)
        for k in range(_NXB):
            sk = nsteps - _NXB + k
            pltpu.make_async_copy(
                bufs[sk % _NXB], o_slice(sk // b, sk % b),
                sos[sk % _NXB]).wait()

    return sc_add


def kernel(x, table):
    B, L, D = x.shape
    out = _make_sc_add(B, L, L, D)(x.reshape(B * L, D), table)
    return out.reshape(B, L, D)
